# Initial kernel scaffold; baseline (speedup 1.0000x reference)
import jax
import jax.numpy as jnp
from jax.experimental import pallas as pl


def kernel(text_embeddings, attribute_embedding, img_token_mask, reference_attribute_num, mlp1_ln_g, mlp1_ln_b, mlp1_w1, mlp1_b1, mlp1_w2, mlp1_b2, mlp2_ln_g, mlp2_ln_b, mlp2_w1, mlp2_b1, mlp2_w2, mlp2_b2, final_ln_g, final_ln_b):
    raise NotImplementedError("write your pallas kernel here")


# speedup vs baseline: 3.1555x; 3.1555x over previous
import jax
import jax.numpy as jnp
from jax.experimental import pallas as pl
from jax.experimental.pallas import tpu as pltpu

_ROWS = 512
_EPS = 1e-5


def _gelu(x):
    return 0.5 * x * (1.0 + jax.lax.erf(x * 0.7071067811865476))


def _ln(x, g, b):
    m = x.mean(-1, keepdims=True)
    v = ((x - m) ** 2).mean(-1, keepdims=True)
    return (x - m) * jax.lax.rsqrt(v + _EPS) * g + b


def _fused_kernel(x_ref, a_ref, gm_ref, gv_ref,
                  ln1g_ref, ln1b_ref, w1_ref, b1_ref, w2_ref, b2_ref,
                  ln2g_ref, ln2b_ref, w3_ref, b3_ref, w4_ref, b4_ref,
                  lnfg_ref, lnfb_ref, o_ref):
    x = x_ref[...]
    gm = gm_ref[0, 0, :][:, None]
    gv = gv_ref[0, 0, :][:, None]
    img = x * gm
    attr = a_ref[...] * gv

    cat = jnp.concatenate([img, attr], axis=-1)
    cat = _ln(cat, ln1g_ref[0, :], ln1b_ref[0, :])
    h = _gelu(jnp.dot(cat, w1_ref[...], preferred_element_type=jnp.float32)
              + b1_ref[0, :])
    h1 = (jnp.dot(h, w2_ref[...], preferred_element_type=jnp.float32)
          + b2_ref[0, :]) + img

    n2 = _ln(h1, ln2g_ref[0, :], ln2b_ref[0, :])
    h = _gelu(jnp.dot(n2, w3_ref[...], preferred_element_type=jnp.float32)
              + b3_ref[0, :])
    h2 = (jnp.dot(h, w4_ref[...], preferred_element_type=jnp.float32)
          + b4_ref[0, :]) + h1

    fused = _ln(h2, lnfg_ref[0, :], lnfb_ref[0, :])
    o_ref[...] = fused * gm + x * (1.0 - gm)


def kernel(text_embeddings, attribute_embedding, img_token_mask,
           reference_attribute_num,
           mlp1_ln_g, mlp1_ln_b, mlp1_w1, mlp1_b1, mlp1_w2, mlp1_b2,
           mlp2_ln_g, mlp2_ln_b, mlp2_w1, mlp2_b1, mlp2_w2, mlp2_b2,
           final_ln_g, final_ln_b):
    b, s, d = text_embeddings.shape
    maxr, t = attribute_embedding.shape[1], attribute_embedding.shape[2]
    n = b * s
    nb = n // _ROWS

    x = text_embeddings.reshape(n, d)
    a = attribute_embedding.reshape(b * maxr * t, d)

    gate_mask = img_token_mask.reshape(-1).astype(jnp.float32)
    valid = (jnp.arange(maxr)[None, :] < reference_attribute_num[:, None])
    gate_valid = jnp.broadcast_to(valid[:, :, None], (b, maxr, t))
    gate_valid = gate_valid.reshape(-1).astype(jnp.float32)
    gm3 = gate_mask.reshape(nb, 1, _ROWS)
    gv3 = gate_valid.reshape(nb, 1, _ROWS)

    vec = lambda p: p.reshape(1, -1)
    row_spec = pl.BlockSpec((_ROWS, d), lambda i: (i, 0))
    gate_spec = pl.BlockSpec((1, 1, _ROWS), lambda i: (i, 0, 0))
    const2 = lambda arr: pl.BlockSpec(arr.shape, lambda i: (0, 0))

    args = (x, a, gm3, gv3,
            vec(mlp1_ln_g), vec(mlp1_ln_b), mlp1_w1, vec(mlp1_b1),
            mlp1_w2, vec(mlp1_b2),
            vec(mlp2_ln_g), vec(mlp2_ln_b), mlp2_w1, vec(mlp2_b1),
            mlp2_w2, vec(mlp2_b2),
            vec(final_ln_g), vec(final_ln_b))
    in_specs = [row_spec, row_spec, gate_spec, gate_spec] + [
        const2(arr) for arr in args[4:]]

    out = pl.pallas_call(
        _fused_kernel,
        grid=(nb,),
        in_specs=in_specs,
        out_specs=row_spec,
        out_shape=jax.ShapeDtypeStruct((n, d), jnp.float32),
        compiler_params=pltpu.CompilerParams(
            dimension_semantics=("arbitrary",)),
    )(*args)
    return out.reshape(b, s, d)


# speedup vs baseline: 3.9130x; 1.2400x over previous
import jax
import jax.numpy as jnp
from jax.experimental import pallas as pl
from jax.experimental.pallas import tpu as pltpu

_ROWS = 512
_EPS = 1e-5


def _gelu(x):
    return 0.5 * x * (1.0 + jax.lax.erf(x * 0.7071067811865476))


def _fused_kernel(x_ref, a_ref, gm_ref, gv_ref,
                  w1x_ref, w1a_ref, gw1_ref, c1_ref, w2_ref, b2_ref,
                  w3_ref, gw3_ref, c3_ref, w4_ref, b4_ref,
                  lnfg_ref, lnfb_ref, o_ref):
    x = x_ref[...]
    gm = gm_ref[0, 0, :][:, None]
    gv = gv_ref[0, 0, :][:, None]
    img = x * gm
    attr = a_ref[...] * gv

    n1 = 2.0 * img.shape[1]
    s1 = (jnp.sum(img, axis=1, keepdims=True)
          + jnp.sum(attr, axis=1, keepdims=True))
    s2 = (jnp.sum(img * img, axis=1, keepdims=True)
          + jnp.sum(attr * attr, axis=1, keepdims=True))
    m = s1 / n1
    inv = jax.lax.rsqrt(s2 / n1 - m * m + _EPS)
    mm = jnp.dot(img, w1x_ref[...], preferred_element_type=jnp.float32)
    mm += jnp.dot(attr, w1a_ref[...], preferred_element_type=jnp.float32)
    h = _gelu(inv * (mm - m * gw1_ref[0, :]) + c1_ref[0, :])
    h1 = (jnp.dot(h, w2_ref[...], preferred_element_type=jnp.float32)
          + b2_ref[0, :]) + img

    n2 = 1.0 * h1.shape[1]
    t1 = jnp.sum(h1, axis=1, keepdims=True)
    t2 = jnp.sum(h1 * h1, axis=1, keepdims=True)
    m2 = t1 / n2
    inv2 = jax.lax.rsqrt(t2 / n2 - m2 * m2 + _EPS)
    mm2 = jnp.dot(h1, w3_ref[...], preferred_element_type=jnp.float32)
    h = _gelu(inv2 * (mm2 - m2 * gw3_ref[0, :]) + c3_ref[0, :])
    h2 = (jnp.dot(h, w4_ref[...], preferred_element_type=jnp.float32)
          + b4_ref[0, :]) + h1

    u1 = jnp.sum(h2, axis=1, keepdims=True)
    u2 = jnp.sum(h2 * h2, axis=1, keepdims=True)
    m3 = u1 / n2
    inv3 = jax.lax.rsqrt(u2 / n2 - m3 * m3 + _EPS)
    fused = (h2 - m3) * inv3 * lnfg_ref[0, :] + lnfb_ref[0, :]
    o_ref[...] = fused * gm + x * (1.0 - gm)


def kernel(text_embeddings, attribute_embedding, img_token_mask,
           reference_attribute_num,
           mlp1_ln_g, mlp1_ln_b, mlp1_w1, mlp1_b1, mlp1_w2, mlp1_b2,
           mlp2_ln_g, mlp2_ln_b, mlp2_w1, mlp2_b1, mlp2_w2, mlp2_b2,
           final_ln_g, final_ln_b):
    b, s, d = text_embeddings.shape
    maxr, t = attribute_embedding.shape[1], attribute_embedding.shape[2]
    n = b * s
    nb = n // _ROWS

    x = text_embeddings.reshape(n, d)
    a = attribute_embedding.reshape(b * maxr * t, d)

    gate_mask = img_token_mask.reshape(-1).astype(jnp.float32)
    valid = (jnp.arange(maxr)[None, :] < reference_attribute_num[:, None])
    gate_valid = jnp.broadcast_to(valid[:, :, None], (b, maxr, t))
    gate_valid = gate_valid.reshape(-1).astype(jnp.float32)
    gm3 = gate_mask.reshape(nb, 1, _ROWS)
    gv3 = gate_valid.reshape(nb, 1, _ROWS)

    w1s = mlp1_ln_g[:, None] * mlp1_w1
    w1x, w1a = w1s[:d], w1s[d:]
    gw1 = (mlp1_ln_g @ mlp1_w1).reshape(1, -1)
    c1 = (mlp1_ln_b @ mlp1_w1 + mlp1_b1).reshape(1, -1)
    w3s = mlp2_ln_g[:, None] * mlp2_w1
    gw3 = (mlp2_ln_g @ mlp2_w1).reshape(1, -1)
    c3 = (mlp2_ln_b @ mlp2_w1 + mlp2_b1).reshape(1, -1)

    vec = lambda p: p.reshape(1, -1)
    row_spec = pl.BlockSpec((_ROWS, d), lambda i: (i, 0))
    gate_spec = pl.BlockSpec((1, 1, _ROWS), lambda i: (i, 0, 0))
    const2 = lambda arr: pl.BlockSpec(arr.shape, lambda i: (0, 0))

    args = (x, a, gm3, gv3,
            w1x, w1a, gw1, c1, mlp1_w2, vec(mlp1_b2),
            w3s, gw3, c3, mlp2_w2, vec(mlp2_b2),
            vec(final_ln_g), vec(final_ln_b))
    in_specs = [row_spec, row_spec, gate_spec, gate_spec] + [
        const2(arr) for arr in args[4:]]

    out = pl.pallas_call(
        _fused_kernel,
        grid=(nb,),
        in_specs=in_specs,
        out_specs=row_spec,
        out_shape=jax.ShapeDtypeStruct((n, d), jnp.float32),
        compiler_params=pltpu.CompilerParams(
            dimension_semantics=("arbitrary",)),
    )(*args)
    return out.reshape(b, s, d)
